# baseline (device time: 38316 ns/iter reference)
import jax
import jax.numpy as jnp
from jax import lax
from jax.experimental import pallas as pl
from jax.experimental.pallas import tpu as pltpu

N_DEV = 8
GELU_C = 0.7978845608028654


def _gelu(y):
    return 0.5 * y * (1.0 + jnp.tanh(GELU_C * (y + 0.044715 * y * y * y)))


def kernel(x, w_mat):
    m_per, k = x.shape
    _, n = w_mat.shape
    n_per = n // N_DEV
    w3 = w_mat.reshape(k, N_DEV, n_per)

    def body(x_ref, w_ref, out_ref, ybuf, recvbuf, send_sems, recv_sems):
        my = lax.axis_index("i")

        barrier_sem = pltpu.get_barrier_semaphore()
        for d in range(1, N_DEV):
            t = lax.rem(my + d, N_DEV)
            pl.semaphore_signal(
                barrier_sem, inc=1,
                device_id=(t,), device_id_type=pl.DeviceIdType.MESH,
            )
        pl.semaphore_wait(barrier_sem, N_DEV - 1)

        xb = x_ref[:, :].astype(jnp.bfloat16)

        rdmas = []
        for d in range(1, N_DEV):
            t = lax.rem(my + d, N_DEV)
            wc = w_ref[:, t, :].astype(jnp.bfloat16)
            yc = jnp.dot(xb, wc, preferred_element_type=jnp.float32)
            ybuf[t, :, :] = _gelu(yc).astype(jnp.bfloat16)
            rdma = pltpu.make_async_remote_copy(
                src_ref=ybuf.at[t],
                dst_ref=recvbuf.at[my],
                send_sem=send_sems.at[t],
                recv_sem=recv_sems.at[my],
                device_id=(t,),
                device_id_type=pl.DeviceIdType.MESH,
            )
            rdma.start()
            rdmas.append(rdma)

        wc = w_ref[:, my, :].astype(jnp.bfloat16)
        yc = jnp.dot(xb, wc, preferred_element_type=jnp.float32)
        out_ref[pl.ds(my * m_per, m_per), :] = _gelu(yc)

        for d in range(1, N_DEV):
            s = lax.rem(my + N_DEV - d, N_DEV)
            recv = pltpu.make_async_remote_copy(
                src_ref=ybuf.at[s],
                dst_ref=recvbuf.at[s],
                send_sem=send_sems.at[s],
                recv_sem=recv_sems.at[s],
                device_id=(s,),
                device_id_type=pl.DeviceIdType.MESH,
            )
            recv.wait_recv()
            out_ref[pl.ds(s * m_per, m_per), :] = recvbuf[s, :, :].astype(
                jnp.float32
            )

        for rdma in rdmas:
            rdma.wait_send()

    return pl.pallas_call(
        body,
        out_shape=jax.ShapeDtypeStruct((N_DEV * m_per, n_per), jnp.float32),
        in_specs=[
            pl.BlockSpec(memory_space=pltpu.VMEM),
            pl.BlockSpec(memory_space=pltpu.VMEM),
        ],
        out_specs=pl.BlockSpec(memory_space=pltpu.VMEM),
        scratch_shapes=[
            pltpu.VMEM((N_DEV, m_per, n_per), jnp.bfloat16),
            pltpu.VMEM((N_DEV, m_per, n_per), jnp.bfloat16),
            pltpu.SemaphoreType.DMA((N_DEV,)),
            pltpu.SemaphoreType.DMA((N_DEV,)),
        ],
        compiler_params=pltpu.CompilerParams(collective_id=0),
    )(x, w3)


# device time: 20679 ns/iter; 1.8529x vs baseline; 1.8529x over previous
import os

import jax
import jax.numpy as jnp
from jax import lax
from jax.experimental import pallas as pl
from jax.experimental.pallas import tpu as pltpu

N_DEV = 8
GELU_C = 0.7978845608028654

MODE = os.environ.get("KMODE", "full")


def _gelu(y):
    return 0.5 * y * (1.0 + jnp.tanh(GELU_C * (y + 0.044715 * y * y * y)))


def kernel(x, w_mat):
    m_per, k = x.shape
    _, n = w_mat.shape
    n_per = n // N_DEV

    def body(x_ref, w_ref, out_ref, ybuf, recvbuf, send_sems, recv_sems):
        my = lax.axis_index("i")

        if MODE != "compute":
            barrier_sem = pltpu.get_barrier_semaphore()
            for d in range(1, N_DEV):
                t = lax.rem(my + d, N_DEV)
                pl.semaphore_signal(
                    barrier_sem, inc=1,
                    device_id=(t,), device_id_type=pl.DeviceIdType.MESH,
                )

        if MODE == "comm":
            for q in range(N_DEV):
                ybuf[q, :, :] = x_ref[:, :n_per].astype(jnp.bfloat16)
            out_ref[pl.ds(my * m_per, m_per), :] = x_ref[:, :n_per]
        else:
            xb = x_ref[:, :].astype(jnp.bfloat16)
            wb = w_ref[:, :].astype(jnp.bfloat16)
            y = jnp.dot(xb, wb, preferred_element_type=jnp.float32)
            y = _gelu(y)
            for q in range(N_DEV):
                ybuf[q, :, :] = y[:, q * n_per:(q + 1) * n_per].astype(
                    jnp.bfloat16
                )
            out_ref[pl.ds(my * m_per, m_per), :] = lax.dynamic_slice(
                y, (0, my * n_per), (m_per, n_per)
            )

        if MODE == "compute":
            for q in range(N_DEV):
                out_ref[pl.ds(q * m_per, m_per), :] = ybuf[q, :, :].astype(
                    jnp.float32
                )
            return

        pl.semaphore_wait(barrier_sem, N_DEV - 1)

        rdmas = []
        for d in range(1, N_DEV):
            t = lax.rem(my + d, N_DEV)
            rdma = pltpu.make_async_remote_copy(
                src_ref=ybuf.at[t],
                dst_ref=recvbuf.at[my],
                send_sem=send_sems.at[t],
                recv_sem=recv_sems.at[my],
                device_id=(t,),
                device_id_type=pl.DeviceIdType.MESH,
            )
            rdma.start()
            rdmas.append(rdma)

        for d in range(1, N_DEV):
            s = lax.rem(my + N_DEV - d, N_DEV)
            recv = pltpu.make_async_remote_copy(
                src_ref=ybuf.at[s],
                dst_ref=recvbuf.at[s],
                send_sem=send_sems.at[s],
                recv_sem=recv_sems.at[s],
                device_id=(s,),
                device_id_type=pl.DeviceIdType.MESH,
            )
            recv.wait_recv()
            out_ref[pl.ds(s * m_per, m_per), :] = recvbuf[s, :, :].astype(
                jnp.float32
            )

        for rdma in rdmas:
            rdma.wait_send()

    return pl.pallas_call(
        body,
        out_shape=jax.ShapeDtypeStruct((N_DEV * m_per, n_per), jnp.float32),
        in_specs=[
            pl.BlockSpec(memory_space=pltpu.VMEM),
            pl.BlockSpec(memory_space=pltpu.VMEM),
        ],
        out_specs=pl.BlockSpec(memory_space=pltpu.VMEM),
        scratch_shapes=[
            pltpu.VMEM((N_DEV, m_per, n_per), jnp.bfloat16),
            pltpu.VMEM((N_DEV, m_per, n_per), jnp.bfloat16),
            pltpu.SemaphoreType.DMA((N_DEV,)),
            pltpu.SemaphoreType.DMA((N_DEV,)),
        ],
        compiler_params=pltpu.CompilerParams(
            collective_id=None if MODE == "compute" else 0
        ),
    )(x, w_mat)
